# baseline (speedup 1.0000x reference)
import functools

import jax
import jax.numpy as jnp
from jax import lax
from jax.experimental import pallas as pl
from jax.experimental.pallas import tpu as pltpu
from jax.experimental.pallas import tpu_sc as plsc

BATCH = 4096
SEQ = 200
D = 64
VOCAB = 1000000
NW = 32
BT = BATCH // NW


def _make_gather():
  mesh = plsc.VectorSubcoreMesh(core_axis_name="c", subcore_axis_name="s")

  @functools.partial(
      pl.kernel,
      mesh=mesh,
      out_type=jax.ShapeDtypeStruct((SEQ, D // 8, NW, 8, BT), jnp.float32),
      scratch_types=[
          pltpu.VMEM((SEQ, BT), jnp.int32),
          pltpu.VMEM((BT, D), jnp.float32),
          pltpu.VMEM((BT, D), jnp.float32),
          pltpu.VMEM((D, BT), jnp.float32),
          pltpu.VMEM((D, BT), jnp.float32),
          pltpu.SemaphoreType.DMA,
          pltpu.SemaphoreType.DMA,
          pltpu.SemaphoreType.DMA,
          pltpu.SemaphoreType.DMA,
      ],
      compiler_params=pltpu.CompilerParams(
          use_tc_tiling_on_sc=False, needs_layout_passes=False),
  )
  def k(idx_hbm, table_hbm, out_hbm, idx_v, rows0, rows1, trsp0, trsp1,
        gsem0, gsem1, ssem0, ssem1):
    rows = (rows0, rows1)
    trsp = (trsp0, trsp1)
    gsem = (gsem0, gsem1)
    ssem = (ssem0, ssem1)
    wid = lax.axis_index("s") * 2 + lax.axis_index("c")

    pltpu.sync_copy(idx_hbm.at[wid], idx_v)

    iota16 = lax.iota(jnp.int32, 16)
    bidx = [iota16 + (bg * 16) for bg in range(BT // 16)]

    def start_gather(l, b):
      pltpu.async_copy(table_hbm.at[idx_v.at[l]], rows[b], gsem[b])

    def wait_gather(l, b):
      pltpu.make_async_copy(table_hbm.at[idx_v.at[l]], rows[b],
                            gsem[b]).wait()

    def transpose(b):
      def body_c(c, carry):
        cidx = jnp.full((16,), c, jnp.int32)
        for bg in range(BT // 16):
          val = plsc.load_gather(rows[b], [bidx[bg], cidx])
          trsp[b][c, pl.ds(bg * 16, 16)] = val
        return carry

      lax.fori_loop(0, D, body_c, 0)

    def start_stores(l, b):
      for ct in range(D // 8):
        pltpu.async_copy(trsp[b].at[pl.ds(ct * 8, 8)],
                         out_hbm.at[l, ct, wid], ssem[b])

    def wait_stores(l, b):
      for ct in range(D // 8):
        pltpu.make_async_copy(trsp[b].at[pl.ds(ct * 8, 8)],
                              out_hbm.at[l, ct, wid], ssem[b]).wait()

    for b in range(2):
      start_gather(b, b)
    for b in range(2):
      wait_gather(b, b)
      transpose(b)
      start_stores(b, b)
      start_gather(b + 2, b)

    def body(i, carry):
      for b in range(2):
        l = i * 2 + b
        wait_gather(l, b)
        wait_stores(l - 2, b)
        transpose(b)
        start_stores(l, b)
        start_gather(l + 2, b)
      return carry

    lax.fori_loop(1, SEQ // 2 - 1, body, 0)

    for b in range(2):
      l = SEQ - 2 + b
      wait_gather(l, b)
      wait_stores(l - 2, b)
      transpose(b)
      start_stores(l, b)
    for b in range(2):
      wait_stores(SEQ - 2 + b, b)

  return k


_gather = _make_gather()


@jax.jit
def kernel(idx, table):
  idx_prep = idx.reshape(NW, BT, SEQ).transpose(0, 2, 1).astype(jnp.int32)
  out5 = _gather(idx_prep, table)
  return out5.transpose(2, 4, 0, 1, 3).reshape(BATCH, SEQ, D)
